# baseline (speedup 1.0000x reference)
import jax
import jax.numpy as jnp
from jax import lax
from jax.experimental import pallas as pl
from jax.experimental.pallas import tpu as pltpu
from jax.experimental.pallas import tpu_sc as plsc

N = 100000
D = 64
CHUNK = 80
VALID_CHUNKS = N // CHUNK
NC = 2
NS = 16
NW = NC * NS
CPW = 40
NBUF = 4
L = 16
SCALE = 1.0 / (N * D)


def _sc_body(i2t_hbm, nmt_hbm, idx0_hbm, idx1_hbm, out_hbm,
             stage0_v, stage1_v, acc_v, idx0_bufs, idx1_bufs,
             a_bufs, b_bufs, sems):
    wid = lax.axis_index("s") * NC + lax.axis_index("c")
    base_chunk = jnp.minimum(wid * CPW, VALID_CHUNKS - CPW)

    pltpu.sync_copy(idx0_hbm.at[pl.ds(base_chunk * CHUNK, CPW * CHUNK)],
                    stage0_v)
    pltpu.sync_copy(idx1_hbm.at[pl.ds(base_chunk * CHUNK, CPW * CHUNK)],
                    stage1_v)

    def prep_issue(c, b):
        g = wid * CPW + c
        gc = jnp.minimum(g, VALID_CHUNKS - 1)
        off = (gc - base_chunk) * CHUNK
        for t in range(CHUNK // L):
            idx0_bufs[b][pl.ds(t * L, L)] = stage0_v[pl.ds(off + t * L, L)]
            idx1_bufs[b][pl.ds(t * L, L)] = stage1_v[pl.ds(off + t * L, L)]
        pltpu.async_copy(i2t_hbm.at[idx0_bufs[b]], a_bufs[b], sems[b])
        pltpu.async_copy(nmt_hbm.at[idx1_bufs[b]], b_bufs[b], sems[b])

    def wait(b):
        pltpu.make_async_copy(i2t_hbm.at[idx0_bufs[b]], a_bufs[b], sems[b]).wait()
        pltpu.make_async_copy(nmt_hbm.at[idx1_bufs[b]], b_bufs[b], sems[b]).wait()

    for b in range(NBUF):
        prep_issue(b, b)

    def outer_body(k, accs):
        for b in range(NBUF):
            c = k * NBUF + b
            wait(b)
            a_v, b_v = a_bufs[b], b_bufs[b]

            def row_body(i, ch):
                out = list(ch)
                for r in range(4):
                    row = i * 4 + r
                    for j in range(4):
                        d = (a_v[row, pl.ds(j * L, L)]
                             - b_v[row, pl.ds(j * L, L)])
                        out[j] = out[j] + d * d
                return tuple(out)

            zero = jnp.zeros((L,), jnp.float32)
            ch = lax.fori_loop(0, CHUNK // 4, row_body,
                               (zero, zero, zero, zero))
            w = jnp.where(wid * CPW + c < VALID_CHUNKS,
                          jnp.float32(1.0), jnp.float32(0.0))
            accs = tuple(t + w * p for t, p in zip(accs, ch))

            @pl.when(c + NBUF < CPW)
            def _():
                prep_issue(c + NBUF, b)
        return accs

    zero = jnp.zeros((L,), jnp.float32)
    accs = lax.fori_loop(0, CPW // NBUF, outer_body, (zero,) * 4)
    total = (accs[0] + accs[1]) + (accs[2] + accs[3])
    acc_v[...] = total * jnp.float32(SCALE)
    pltpu.sync_copy(acc_v, out_hbm.at[wid])


@jax.jit
def _sc_mse(i2t_wemb, nmt_wemb, idx0, idx1):
    mesh = plsc.VectorSubcoreMesh(core_axis_name="c", subcore_axis_name="s",
                                  num_cores=NC, num_subcores=NS)
    f = pl.kernel(
        _sc_body,
        out_type=jax.ShapeDtypeStruct((NW, L), jnp.float32),
        mesh=mesh,
        scratch_types=[
            pltpu.VMEM((CPW * CHUNK,), jnp.int32),
            pltpu.VMEM((CPW * CHUNK,), jnp.int32),
            pltpu.VMEM((L,), jnp.float32),
            [pltpu.VMEM((CHUNK,), jnp.int32) for _ in range(NBUF)],
            [pltpu.VMEM((CHUNK,), jnp.int32) for _ in range(NBUF)],
            [pltpu.VMEM((CHUNK, 2 * D), jnp.float32) for _ in range(NBUF)],
            [pltpu.VMEM((CHUNK, D), jnp.float32) for _ in range(NBUF)],
            [pltpu.SemaphoreType.DMA for _ in range(NBUF)],
        ],
        compiler_params=pltpu.CompilerParams(use_tc_tiling_on_sc=False,
                                             needs_layout_passes=False),
    )
    return f(i2t_wemb, nmt_wemb, idx0, idx1)


def kernel(maps, i2t_wemb, nmt_wemb):
    i2t_pad = jnp.pad(i2t_wemb, ((0, 0), (0, D)))
    partials = _sc_mse(i2t_pad, nmt_wemb, maps[:, 0], maps[:, 1])
    return jnp.sum(partials)


# speedup vs baseline: 1.0170x; 1.0170x over previous
import jax
import jax.numpy as jnp
from jax import lax
from jax.experimental import pallas as pl
from jax.experimental.pallas import tpu as pltpu
from jax.experimental.pallas import tpu_sc as plsc

N = 100000
D = 64
CHUNK = 80
VALID_CHUNKS = N // CHUNK
NC = 2
NS = 16
NW = NC * NS
CPW = 40
NBUF = 5
L = 16
SCALE = 1.0 / (N * D)


def _sc_body(i2t_hbm, nmt_hbm, idx0_hbm, idx1_hbm, out_hbm,
             stage0_v, stage1_v, acc_v, a_bufs, b_bufs, sems):
    wid = lax.axis_index("s") * NC + lax.axis_index("c")
    base_chunk = jnp.minimum(wid * CPW, VALID_CHUNKS - CPW)

    pltpu.sync_copy(idx0_hbm.at[pl.ds(base_chunk * CHUNK, CPW * CHUNK)],
                    stage0_v)
    pltpu.sync_copy(idx1_hbm.at[pl.ds(base_chunk * CHUNK, CPW * CHUNK)],
                    stage1_v)

    def chunk_off(c):
        g = wid * CPW + c
        gc = jnp.minimum(g, VALID_CHUNKS - 1)
        return (gc - base_chunk) * CHUNK

    def issue(c, b):
        off = chunk_off(c)
        pltpu.async_copy(i2t_hbm.at[stage0_v.at[pl.ds(off, CHUNK)]],
                         a_bufs[b], sems[b])
        pltpu.async_copy(nmt_hbm.at[stage1_v.at[pl.ds(off, CHUNK)]],
                         b_bufs[b], sems[b])

    def wait(b):
        off0 = chunk_off(0)
        pltpu.make_async_copy(i2t_hbm.at[stage0_v.at[pl.ds(off0, CHUNK)]],
                              a_bufs[b], sems[b]).wait()
        pltpu.make_async_copy(nmt_hbm.at[stage1_v.at[pl.ds(off0, CHUNK)]],
                              b_bufs[b], sems[b]).wait()

    for b in range(NBUF):
        issue(b, b)

    def outer_body(k, accs):
        for b in range(NBUF):
            c = k * NBUF + b
            wait(b)
            a_v, b_v = a_bufs[b], b_bufs[b]

            def row_body(i, ch):
                out = list(ch)
                for r in range(4):
                    row = i * 4 + r
                    for j in range(4):
                        d = (a_v[row, pl.ds(j * L, L)]
                             - b_v[row, pl.ds(j * L, L)])
                        out[j] = out[j] + d * d
                return tuple(out)

            zero = jnp.zeros((L,), jnp.float32)
            ch = lax.fori_loop(0, CHUNK // 4, row_body,
                               (zero, zero, zero, zero))
            w = jnp.where(wid * CPW + c < VALID_CHUNKS,
                          jnp.float32(1.0), jnp.float32(0.0))
            accs = tuple(t + w * p for t, p in zip(accs, ch))

            @pl.when(c + NBUF < CPW)
            def _():
                issue(c + NBUF, b)
        return accs

    zero = jnp.zeros((L,), jnp.float32)
    accs = lax.fori_loop(0, CPW // NBUF, outer_body, (zero,) * 4)
    total = (accs[0] + accs[1]) + (accs[2] + accs[3])
    acc_v[...] = total * jnp.float32(SCALE)
    pltpu.sync_copy(acc_v, out_hbm.at[wid])


@jax.jit
def _sc_mse(i2t_wemb, nmt_wemb, idx0, idx1):
    mesh = plsc.VectorSubcoreMesh(core_axis_name="c", subcore_axis_name="s",
                                  num_cores=NC, num_subcores=NS)
    f = pl.kernel(
        _sc_body,
        out_type=jax.ShapeDtypeStruct((NW, L), jnp.float32),
        mesh=mesh,
        scratch_types=[
            pltpu.VMEM((CPW * CHUNK,), jnp.int32),
            pltpu.VMEM((CPW * CHUNK,), jnp.int32),
            pltpu.VMEM((L,), jnp.float32),
            [pltpu.VMEM((CHUNK, D), jnp.float32) for _ in range(NBUF)],
            [pltpu.VMEM((CHUNK, D), jnp.float32) for _ in range(NBUF)],
            [pltpu.SemaphoreType.DMA for _ in range(NBUF)],
        ],
        compiler_params=pltpu.CompilerParams(use_tc_tiling_on_sc=False,
                                             needs_layout_passes=False),
    )
    return f(i2t_wemb, nmt_wemb, idx0, idx1)


def kernel(maps, i2t_wemb, nmt_wemb):
    partials = _sc_mse(i2t_wemb, nmt_wemb, maps[:, 0], maps[:, 1])
    return jnp.sum(partials)
